# baseline (speedup 1.0000x reference)
import functools

import jax
import jax.numpy as jnp
from jax.experimental import pallas as pl
from jax.experimental.pallas import tpu as pltpu


def _t2_build_body(w_ref, o_ref):
    o_ref[...] = jnp.zeros_like(o_ref)
    blk = w_ref[...]
    for ph in range(2):
        for j in range(56):
            rs = 6 * (2 * j + ph)
            ls = 896 * ph + 16 * j
            o_ref[:, rs:rs + 30, ls:ls + 16] = blk


def _toep2(w_taps):
    w5 = w_taps.reshape(5, 30, 16)
    return pl.pallas_call(
        _t2_build_body,
        out_shape=jax.ShapeDtypeStruct((5, 696, 1792), jnp.float32),
        in_specs=[pl.BlockSpec(memory_space=pltpu.VMEM)],
        out_specs=pl.BlockSpec(memory_space=pltpu.VMEM),
        compiler_params=pltpu.CompilerParams(
            vmem_limit_bytes=48 * 1024 * 1024),
    )(w5)


def _t1_build_body(w_ref, o_ref):
    o_ref[...] = jnp.zeros_like(o_ref)
    blk = w_ref[...]
    for ph in range(2):
        for j in range(112):
            w0 = max(0, 2 * j + ph - 2)
            w1 = min(224, 2 * j + ph + 3)
            u0 = w0 - (2 * j + ph - 2)
            ls = 768 * ph + 6 * j
            o_ref[:, :, w0:w1, ls:ls + 6] = blk[:, :, u0:u0 + w1 - w0, :]


def _toep1(w_taps):
    w5c = w_taps.reshape(5, 5, 3, 6).transpose(2, 0, 1, 3)
    return pl.pallas_call(
        _t1_build_body,
        out_shape=jax.ShapeDtypeStruct((3, 5, 224, 1536), jnp.float32),
        in_specs=[pl.BlockSpec(memory_space=pltpu.VMEM)],
        out_specs=pl.BlockSpec(memory_space=pltpu.VMEM),
        compiler_params=pltpu.CompilerParams(
            vmem_limit_bytes=48 * 1024 * 1024),
    )(w5c)


def _t3_build_body(w_ref, o_ref):
    j = pl.program_id(0)
    o_ref[...] = jnp.zeros_like(o_ref)
    base = pl.multiple_of(16 * (2 * j + 1), 16)
    o_ref[:, pl.ds(base, 80), :] = w_ref[...]


def _toep3(w_taps):
    w5p = jnp.pad(w_taps.reshape(5, 5, 16, 120),
                  ((0, 0), (0, 0), (0, 0), (0, 8))).reshape(5, 80, 128)
    return pl.pallas_call(
        _t3_build_body,
        out_shape=jax.ShapeDtypeStruct((5, 976, 3584), jnp.float32),
        grid=(28,),
        in_specs=[pl.BlockSpec(memory_space=pltpu.VMEM)],
        out_specs=pl.BlockSpec((5, 976, 128), lambda j: (0, 0, j)),
        compiler_params=pltpu.CompilerParams(
            dimension_semantics=("parallel",),
            vmem_limit_bytes=48 * 1024 * 1024),
    )(w5p)


def _c12_body(x_ref, t1_ref, t2_ref, b1_ref, b2_ref, o_ref, acc1, y1, acc2):
    acc1[...] = jnp.zeros_like(acc1)
    for kh in range(5):
        lo = max(0, 2 - kh)
        hi = min(224, 226 - kh)
        sl = lo + kh - 2
        for c in range(3):
            z = jnp.dot(x_ref[c], t1_ref[c, kh],
                        preferred_element_type=jnp.float32)
            acc1[lo:hi] += z[sl:sl + hi - lo]
    a1 = jnp.maximum(acc1[...] + b1_ref[...], 0.0)
    pw1 = jnp.maximum(a1[:, :768], a1[:, 768:])
    pooled1 = pw1.reshape(112, 2, 768).max(axis=1)
    y1[...] = jnp.zeros_like(y1)
    y1[2:114, 12:684] = pooled1[:, :672]

    for kh in range(5):
        z = jnp.dot(y1[kh:kh + 112, :], t2_ref[kh],
                    preferred_element_type=jnp.float32)
        if kh == 0:
            acc2[...] = z
        else:
            acc2[...] += z
    a2 = jnp.maximum(acc2[...] + b2_ref[...], 0.0)
    pw2 = jnp.maximum(a2[:, :896], a2[:, 896:])
    o_ref[...] = jnp.zeros_like(o_ref)
    o_ref[2:58, 32:928] = pw2.reshape(56, 2, 896).max(axis=1)


def _conv12f(x_nchw, c1w, c1b, c2w, c2b):
    n = x_nchw.shape[0]
    t1 = _toep1(c1w)
    t2 = _toep2(c2w)
    b1t = jnp.tile(jnp.pad(jnp.tile(c1b, (1, 112)), ((0, 0), (0, 96))), (1, 2))
    b2t = jnp.tile(c2b, (1, 112))
    return pl.pallas_call(
        _c12_body,
        out_shape=jax.ShapeDtypeStruct((n, 64, 960), jnp.float32),
        grid=(n,),
        in_specs=[
            pl.BlockSpec((None, 3, 224, 224), lambda ni: (ni, 0, 0, 0)),
            pl.BlockSpec(memory_space=pltpu.VMEM),
            pl.BlockSpec(memory_space=pltpu.VMEM),
            pl.BlockSpec(memory_space=pltpu.VMEM),
            pl.BlockSpec(memory_space=pltpu.VMEM),
        ],
        out_specs=pl.BlockSpec((None, 64, 960), lambda ni: (ni, 0, 0)),
        scratch_shapes=[pltpu.VMEM((224, 1536), jnp.float32),
                        pltpu.VMEM((116, 696), jnp.float32),
                        pltpu.VMEM((112, 1792), jnp.float32)],
        compiler_params=pltpu.CompilerParams(
            dimension_semantics=("parallel",),
            vmem_limit_bytes=56 * 1024 * 1024),
    )(x_nchw, t1, t2, b1t, b2t)


def _c1_body(x_ref, t_ref, b_ref, o_ref, acc_e, acc_o):
    acc_e[...] = jnp.zeros_like(acc_e)
    acc_o[...] = jnp.zeros_like(acc_o)
    for kh in range(5):
        lo = max(0, 2 - kh)
        hi = min(224, 226 - kh)
        sl = lo + kh - 2
        for c in range(3):
            z = jnp.dot(x_ref[c], t_ref[c, kh],
                        preferred_element_type=jnp.float32)
            acc_e[lo:hi] += z[sl:sl + hi - lo, :768]
            acc_o[lo:hi] += z[sl:sl + hi - lo, 768:]
    bias = b_ref[...]
    ae = jnp.maximum(acc_e[...] + bias, 0.0)
    ao = jnp.maximum(acc_o[...] + bias, 0.0)
    pw = jnp.maximum(ae, ao)
    pooled = pw.reshape(112, 2, 768).max(axis=1)
    o_ref[...] = jnp.zeros_like(o_ref)
    o_ref[2:114, 12:684] = pooled[:, :672]


def _conv1(x_nchw, w_taps, b_row):
    n = x_nchw.shape[0]
    t1 = _toep1(w_taps)
    bt = jnp.pad(jnp.tile(b_row, (1, 112)), ((0, 0), (0, 96)))
    return pl.pallas_call(
        _c1_body,
        out_shape=jax.ShapeDtypeStruct((n, 116, 696), jnp.float32),
        grid=(n,),
        in_specs=[
            pl.BlockSpec((None, 3, 224, 224), lambda ni: (ni, 0, 0, 0)),
            pl.BlockSpec(memory_space=pltpu.VMEM),
            pl.BlockSpec(memory_space=pltpu.VMEM),
        ],
        out_specs=pl.BlockSpec((None, 116, 696), lambda ni: (ni, 0, 0)),
        scratch_shapes=[pltpu.VMEM((224, 768), jnp.float32),
                        pltpu.VMEM((224, 768), jnp.float32)],
        compiler_params=pltpu.CompilerParams(
            dimension_semantics=("parallel",),
            vmem_limit_bytes=48 * 1024 * 1024),
    )(x_nchw, t1, bt)


def _c2_body(x_ref, t_ref, b_ref, o_ref, acc):
    for kh in range(5):
        z = jnp.dot(x_ref[kh:kh + 112, :], t_ref[kh],
                    preferred_element_type=jnp.float32)
        if kh == 0:
            acc[...] = z
        else:
            acc[...] += z
    a = jnp.maximum(acc[...] + b_ref[...], 0.0)
    pw = jnp.maximum(a[:, :896], a[:, 896:])
    pooled = pw.reshape(56, 2, 896).max(axis=1)
    o_ref[...] = jnp.zeros_like(o_ref)
    o_ref[2:58, 32:928] = pooled


def _conv2(xf, w_taps, b_row):
    n = xf.shape[0]
    t2 = _toep2(w_taps)
    bt = jnp.tile(b_row, (1, 112))
    return pl.pallas_call(
        _c2_body,
        out_shape=jax.ShapeDtypeStruct((n, 64, 960), jnp.float32),
        grid=(n,),
        in_specs=[
            pl.BlockSpec((None, 116, 696), lambda ni: (ni, 0, 0)),
            pl.BlockSpec(memory_space=pltpu.VMEM),
            pl.BlockSpec(memory_space=pltpu.VMEM),
        ],
        out_specs=pl.BlockSpec((None, 64, 960), lambda ni: (ni, 0, 0)),
        scratch_shapes=[pltpu.VMEM((112, 1792), jnp.float32)],
        compiler_params=pltpu.CompilerParams(
            dimension_semantics=("parallel",),
            vmem_limit_bytes=48 * 1024 * 1024),
    )(xf, t2, bt)


_C3_IMGS = 8
_C3_ROWS = 64
_C3_NT = 4
_C3_K = 960
_C3_NL = 896


def _c3_body(x_ref, t_ref, b_ref, o_ref, acc_e, acc_o):
    m = _C3_IMGS * _C3_ROWS
    mv = m - _C3_ROWS + 56 + 4
    x2 = x_ref[...].reshape(m, _C3_K)
    mo = mv - 4
    for kh in range(5):
        lhs = x2[kh:kh + mo]
        for sl, acc in ((16, acc_e), (0, acc_o)):
            rhs = t_ref[kh, sl:sl + _C3_K, :]
            z = jnp.dot(lhs, rhs, preferred_element_type=jnp.float32)
            if kh == 0:
                acc[...] = z
            else:
                acc[...] += z
    bias = b_ref[...]
    ae = jnp.maximum(acc_e[...] + bias, 0.0)
    ao = jnp.maximum(acc_o[...] + bias, 0.0)
    pw = jnp.maximum(ae, ao)
    for i in range(_C3_IMGS):
        o_ref[i] = pw[i * _C3_ROWS:i * _C3_ROWS + 56].reshape(28, 2, _C3_NL).max(axis=1)


def _conv3(xf, w_taps, b_row):
    n = xf.shape[0]
    t3 = _toep3(w_taps)
    bt = jnp.tile(jnp.pad(b_row, ((0, 0), (0, 8))), (1, 7))
    return pl.pallas_call(
        _c3_body,
        out_shape=jax.ShapeDtypeStruct((n, 28, 28 * 128), jnp.float32),
        grid=(_C3_NT, n // _C3_IMGS),
        in_specs=[
            pl.BlockSpec((_C3_IMGS, _C3_ROWS, _C3_K), lambda t, ib: (ib, 0, 0)),
            pl.BlockSpec((5, 976, _C3_NL), lambda t, ib: (0, 0, t)),
            pl.BlockSpec((1, _C3_NL), lambda t, ib: (0, 0)),
        ],
        out_specs=pl.BlockSpec((_C3_IMGS, 28, _C3_NL), lambda t, ib: (ib, 0, t)),
        scratch_shapes=[pltpu.VMEM((504, _C3_NL), jnp.float32),
                        pltpu.VMEM((504, _C3_NL), jnp.float32)],
        compiler_params=pltpu.CompilerParams(
            dimension_semantics=("parallel", "arbitrary"),
            vmem_limit_bytes=48 * 1024 * 1024),
    )(xf, t3, bt)


def _mlp_body(x_ref, w1_ref, b1_ref, w2_ref, b2_ref, o_ref, acc_ref):
    k = pl.program_id(0)

    @pl.when(k == 0)
    def _():
        acc_ref[...] = jnp.zeros_like(acc_ref)

    acc_ref[...] += jnp.dot(x_ref[...], w1_ref[...],
                            preferred_element_type=jnp.float32)

    @pl.when(k == pl.num_programs(0) - 1)
    def _():
        h = jnp.maximum(acc_ref[...] + b1_ref[...], 0.0)
        o_ref[...] = jnp.dot(h, w2_ref[...],
                             preferred_element_type=jnp.float32) + b2_ref[...]


def _mlp_head(feats, w1, b1, w2, b2, *, tk):
    n, kdim = feats.shape
    h1 = w1.shape[1]
    o = w2.shape[1]
    return pl.pallas_call(
        _mlp_body,
        out_shape=jax.ShapeDtypeStruct((n, o), jnp.float32),
        grid=(kdim // tk,),
        in_specs=[
            pl.BlockSpec((n, tk), lambda k: (0, k)),
            pl.BlockSpec((tk, h1), lambda k: (k, 0)),
            pl.BlockSpec((1, h1), lambda k: (0, 0)),
            pl.BlockSpec((h1, o), lambda k: (0, 0)),
            pl.BlockSpec((1, o), lambda k: (0, 0)),
        ],
        out_specs=pl.BlockSpec((n, o), lambda k: (0, 0)),
        scratch_shapes=[pltpu.VMEM((n, h1), jnp.float32)],
        compiler_params=pltpu.CompilerParams(
            dimension_semantics=("arbitrary",),
            vmem_limit_bytes=48 * 1024 * 1024),
    )(feats, w1, b1, w2, b2)


def _pad_flat(y, wc):
    return jnp.pad(y, ((0, 0), (2, 2), (2 * wc, 2 * wc)))


def kernel(x_nchw, c1w, c1b, c2w, c2b, c3w, c3b, f1w, f1b, f2w, f2b):
    n = x_nchw.shape[0]
    x3 = _conv12f(x_nchw, c1w, c1b, c2w, c2b)
    y3 = _conv3(x3, c3w, c3b)
    feats = y3.reshape(n, 28, 28, 128)[:, :, :, :120].reshape(n, 94080)
    return _mlp_head(feats, f1w, f1b, f2w, f2b, tk=18816)


# speedup vs baseline: 1.0154x; 1.0154x over previous
import functools

import jax
import jax.numpy as jnp
from jax.experimental import pallas as pl
from jax.experimental.pallas import tpu as pltpu


def _t2_build_body(w_ref, o_ref):
    o_ref[...] = jnp.zeros_like(o_ref)
    blk = w_ref[...]
    for ph in range(2):
        for j in range(56):
            rs = 6 * (2 * j + ph)
            ls = 896 * ph + 16 * j
            o_ref[:, rs:rs + 30, ls:ls + 16] = blk


def _toep2(w_taps):
    w5 = w_taps.reshape(5, 30, 16)
    return pl.pallas_call(
        _t2_build_body,
        out_shape=jax.ShapeDtypeStruct((5, 696, 1792), jnp.float32),
        in_specs=[pl.BlockSpec(memory_space=pltpu.VMEM)],
        out_specs=pl.BlockSpec(memory_space=pltpu.VMEM),
        compiler_params=pltpu.CompilerParams(
            vmem_limit_bytes=48 * 1024 * 1024),
    )(w5)


def _t1_build_body(w_ref, o_ref):
    o_ref[...] = jnp.zeros_like(o_ref)
    blk = w_ref[...]
    for ph in range(2):
        for j in range(112):
            w0 = max(0, 2 * j + ph - 2)
            w1 = min(224, 2 * j + ph + 3)
            u0 = w0 - (2 * j + ph - 2)
            ls = 768 * ph + 6 * j
            o_ref[:, :, w0:w1, ls:ls + 6] = blk[:, :, u0:u0 + w1 - w0, :]


def _toep1(w_taps):
    w5c = w_taps.reshape(5, 5, 3, 6).transpose(2, 0, 1, 3)
    return pl.pallas_call(
        _t1_build_body,
        out_shape=jax.ShapeDtypeStruct((3, 5, 224, 1536), jnp.float32),
        in_specs=[pl.BlockSpec(memory_space=pltpu.VMEM)],
        out_specs=pl.BlockSpec(memory_space=pltpu.VMEM),
        compiler_params=pltpu.CompilerParams(
            vmem_limit_bytes=48 * 1024 * 1024),
    )(w5c)


def _t3_build_body(w_ref, o_ref):
    j = pl.program_id(0)
    o_ref[...] = jnp.zeros_like(o_ref)
    base = pl.multiple_of(16 * (2 * j + 1), 16)
    o_ref[:, pl.ds(base, 80), :] = w_ref[...]


def _toep3(w_taps):
    w5p = jnp.pad(w_taps.reshape(5, 5, 16, 120),
                  ((0, 0), (0, 0), (0, 0), (0, 8))).reshape(5, 80, 128)
    return pl.pallas_call(
        _t3_build_body,
        out_shape=jax.ShapeDtypeStruct((5, 976, 3584), jnp.float32),
        grid=(28,),
        in_specs=[pl.BlockSpec(memory_space=pltpu.VMEM)],
        out_specs=pl.BlockSpec((5, 976, 128), lambda j: (0, 0, j)),
        compiler_params=pltpu.CompilerParams(
            dimension_semantics=("parallel",),
            vmem_limit_bytes=48 * 1024 * 1024),
    )(w5p)


def _c1_body(x_ref, t_ref, b_ref, o_ref, acc_e, acc_o):
    acc_e[...] = jnp.zeros_like(acc_e)
    acc_o[...] = jnp.zeros_like(acc_o)
    for kh in range(5):
        lo = max(0, 2 - kh)
        hi = min(224, 226 - kh)
        sl = lo + kh - 2
        for c in range(3):
            z = jnp.dot(x_ref[c], t_ref[c, kh],
                        preferred_element_type=jnp.float32)
            acc_e[lo:hi] += z[sl:sl + hi - lo, :768]
            acc_o[lo:hi] += z[sl:sl + hi - lo, 768:]
    bias = b_ref[...]
    ae = jnp.maximum(acc_e[...] + bias, 0.0)
    ao = jnp.maximum(acc_o[...] + bias, 0.0)
    pw = jnp.maximum(ae, ao)
    pooled = pw.reshape(112, 2, 768).max(axis=1)
    o_ref[...] = jnp.zeros_like(o_ref)
    o_ref[2:114, 12:684] = pooled[:, :672]


def _conv1(x_nchw, w_taps, b_row):
    n = x_nchw.shape[0]
    t1 = _toep1(w_taps)
    bt = jnp.pad(jnp.tile(b_row, (1, 112)), ((0, 0), (0, 96)))
    return pl.pallas_call(
        _c1_body,
        out_shape=jax.ShapeDtypeStruct((n, 116, 696), jnp.float32),
        grid=(n,),
        in_specs=[
            pl.BlockSpec((None, 3, 224, 224), lambda ni: (ni, 0, 0, 0)),
            pl.BlockSpec(memory_space=pltpu.VMEM),
            pl.BlockSpec(memory_space=pltpu.VMEM),
        ],
        out_specs=pl.BlockSpec((None, 116, 696), lambda ni: (ni, 0, 0)),
        scratch_shapes=[pltpu.VMEM((224, 768), jnp.float32),
                        pltpu.VMEM((224, 768), jnp.float32)],
        compiler_params=pltpu.CompilerParams(
            dimension_semantics=("parallel",),
            vmem_limit_bytes=48 * 1024 * 1024),
    )(x_nchw, t1, bt)


def _c2_body(x_ref, t_ref, b_ref, o_ref, acc):
    for kh in range(5):
        z = jnp.dot(x_ref[kh:kh + 112, :], t_ref[kh],
                    preferred_element_type=jnp.float32)
        if kh == 0:
            acc[...] = z
        else:
            acc[...] += z
    a = jnp.maximum(acc[...] + b_ref[...], 0.0)
    pw = jnp.maximum(a[:, :896], a[:, 896:])
    pooled = pw.reshape(56, 2, 896).max(axis=1)
    o_ref[...] = jnp.zeros_like(o_ref)
    o_ref[2:58, 32:928] = pooled


def _conv2(xf, w_taps, b_row):
    n = xf.shape[0]
    t2 = _toep2(w_taps)
    bt = jnp.tile(b_row, (1, 112))
    return pl.pallas_call(
        _c2_body,
        out_shape=jax.ShapeDtypeStruct((n, 64, 960), jnp.float32),
        grid=(n,),
        in_specs=[
            pl.BlockSpec((None, 116, 696), lambda ni: (ni, 0, 0)),
            pl.BlockSpec(memory_space=pltpu.VMEM),
            pl.BlockSpec(memory_space=pltpu.VMEM),
        ],
        out_specs=pl.BlockSpec((None, 64, 960), lambda ni: (ni, 0, 0)),
        scratch_shapes=[pltpu.VMEM((112, 1792), jnp.float32)],
        compiler_params=pltpu.CompilerParams(
            dimension_semantics=("parallel",),
            vmem_limit_bytes=48 * 1024 * 1024),
    )(xf, t2, bt)


_C3_IMGS = 8
_C3_ROWS = 64
_C3_NT = 4
_C3_K = 960
_C3_NL = 896


def _c3_body(x_ref, t_ref, b_ref, o_ref, acc_e, acc_o):
    m = _C3_IMGS * _C3_ROWS
    mv = m - _C3_ROWS + 56 + 4
    x2 = x_ref[...].reshape(m, _C3_K)
    mo = mv - 4
    for kh in range(5):
        lhs = x2[kh:kh + mo]
        for sl, acc in ((16, acc_e), (0, acc_o)):
            rhs = t_ref[kh, sl:sl + _C3_K, :]
            z = jnp.dot(lhs, rhs, preferred_element_type=jnp.float32)
            if kh == 0:
                acc[...] = z
            else:
                acc[...] += z
    bias = b_ref[...]
    ae = jnp.maximum(acc_e[...] + bias, 0.0)
    ao = jnp.maximum(acc_o[...] + bias, 0.0)
    pw = jnp.maximum(ae, ao)
    for i in range(_C3_IMGS):
        o_ref[i] = pw[i * _C3_ROWS:i * _C3_ROWS + 56].reshape(28, 2, _C3_NL).max(axis=1)


def _conv3(xf, w_taps, b_row):
    n = xf.shape[0]
    t3 = _toep3(w_taps)
    bt = jnp.tile(jnp.pad(b_row, ((0, 0), (0, 8))), (1, 7))
    return pl.pallas_call(
        _c3_body,
        out_shape=jax.ShapeDtypeStruct((n, 28, 28 * 128), jnp.float32),
        grid=(_C3_NT, n // _C3_IMGS),
        in_specs=[
            pl.BlockSpec((_C3_IMGS, _C3_ROWS, _C3_K), lambda t, ib: (ib, 0, 0)),
            pl.BlockSpec((5, 976, _C3_NL), lambda t, ib: (0, 0, t)),
            pl.BlockSpec((1, _C3_NL), lambda t, ib: (0, 0)),
        ],
        out_specs=pl.BlockSpec((_C3_IMGS, 28, _C3_NL), lambda t, ib: (ib, 0, t)),
        scratch_shapes=[pltpu.VMEM((504, _C3_NL), jnp.float32),
                        pltpu.VMEM((504, _C3_NL), jnp.float32)],
        compiler_params=pltpu.CompilerParams(
            dimension_semantics=("parallel", "arbitrary"),
            vmem_limit_bytes=48 * 1024 * 1024),
    )(xf, t3, bt)


def _mlp_body(x_ref, w1_ref, b1_ref, w2_ref, b2_ref, o_ref, acc_ref):
    k = pl.program_id(0)

    @pl.when(k == 0)
    def _():
        acc_ref[...] = jnp.zeros_like(acc_ref)

    acc_ref[...] += jnp.dot(x_ref[...], w1_ref[...],
                            preferred_element_type=jnp.float32)

    @pl.when(k == pl.num_programs(0) - 1)
    def _():
        h = jnp.maximum(acc_ref[...] + b1_ref[...], 0.0)
        o_ref[...] = jnp.dot(h, w2_ref[...],
                             preferred_element_type=jnp.float32) + b2_ref[...]


def _mlp_head(feats, w1, b1, w2, b2, *, tk):
    n, kdim = feats.shape
    h1 = w1.shape[1]
    o = w2.shape[1]
    return pl.pallas_call(
        _mlp_body,
        out_shape=jax.ShapeDtypeStruct((n, o), jnp.float32),
        grid=(kdim // tk,),
        in_specs=[
            pl.BlockSpec((n, tk), lambda k: (0, k)),
            pl.BlockSpec((tk, h1), lambda k: (k, 0)),
            pl.BlockSpec((1, h1), lambda k: (0, 0)),
            pl.BlockSpec((h1, o), lambda k: (0, 0)),
            pl.BlockSpec((1, o), lambda k: (0, 0)),
        ],
        out_specs=pl.BlockSpec((n, o), lambda k: (0, 0)),
        scratch_shapes=[pltpu.VMEM((n, h1), jnp.float32)],
        compiler_params=pltpu.CompilerParams(
            dimension_semantics=("arbitrary",),
            vmem_limit_bytes=48 * 1024 * 1024),
    )(feats, w1, b1, w2, b2)


def _pad_flat(y, wc):
    return jnp.pad(y, ((0, 0), (2, 2), (2 * wc, 2 * wc)))


def kernel(x_nchw, c1w, c1b, c2w, c2b, c3w, c3b, f1w, f1b, f2w, f2b):
    n = x_nchw.shape[0]
    x2 = _conv1(x_nchw, c1w, c1b)
    x3 = _conv2(x2, c2w, c2b)
    y3 = _conv3(x3, c3w, c3b)
    feats = y3.reshape(n, 28, 28, 128)[:, :, :, :120].reshape(n, 94080)
    return _mlp_head(feats, f1w, f1b, f2w, f2b, tk=18816)
